# baseline (speedup 1.0000x reference)
import jax
import jax.numpy as jnp
from jax.experimental import pallas as pl
from jax.experimental.pallas import tpu as pltpu


_ROWS = 512


def _add_window_kernel(t8_ref, g3_ref, out_ref):
    out_ref[...] = t8_ref[...]

    @pl.when(pl.program_id(0) % (1024 // _ROWS) == 0)
    def _():
        for h in range(12):
            out_ref[1:256, h * 513 + 1 : h * 513 + 256] += g3_ref[0, h, :, :]


def kernel(transpose_8, getitem_3, view_4):
    del view_4
    g3 = getitem_3.reshape(4, 12, 255, 255)
    t8 = transpose_8.reshape(4096, 6156)
    per_b = 1024 // _ROWS
    out = pl.pallas_call(
        _add_window_kernel,
        grid=(4096 // _ROWS,),
        in_specs=[
            pl.BlockSpec((_ROWS, 6156), lambda i: (i, 0)),
            pl.BlockSpec((1, 12, 255, 255), lambda i: (i // per_b, 0, 0, 0)),
        ],
        out_specs=pl.BlockSpec((_ROWS, 6156), lambda i: (i, 0)),
        out_shape=jax.ShapeDtypeStruct((4096, 6156), transpose_8.dtype),
        compiler_params=pltpu.CompilerParams(
            dimension_semantics=("arbitrary",)
        ),
    )(t8, g3)
    return (out.reshape(4, 1024, 12, 513),)
